# Initial kernel scaffold; baseline (speedup 1.0000x reference)
import jax
import jax.numpy as jnp
from jax.experimental import pallas as pl


def kernel(x, edge_index, Wl0, bl0, Wr0, br0, Wl1, bl1, Wr1, br1, Wlo, blo, Wro, bro):
    raise NotImplementedError("write your pallas kernel here")


# speedup vs baseline: 4.7911x; 4.7911x over previous
import functools

import jax
import jax.numpy as jnp
from jax import lax
from jax.experimental import pallas as pl
from jax.experimental.pallas import tpu as pltpu
from jax.experimental.pallas import tpu_sc as plsc

N = 10000
D = 128
E = 320000
C = 2

NC = 2
NS = 16
NW = NC * NS
EPW = E // NW
CH = 128
NFULL = EPW // CH
TAIL = EPW - NFULL * CH
NCHUNK = NFULL + 1
SRCPAD = NCHUNK * CH
NPAD = 10240
RPS = NPAD // NS


def _sc_agg(d, compute_deg):
    scratch = [
        pltpu.VMEM((SRCPAD,), jnp.int32),
        pltpu.VMEM((NCHUNK, CH), jnp.int32),
        pltpu.VMEM((CH, d), jnp.float32),
        pltpu.VMEM_SHARED((NPAD, d), jnp.float32),
        pltpu.SemaphoreType.DMA,
    ]
    out_type = [jax.ShapeDtypeStruct((NC, NPAD, d), jnp.float32)]
    if compute_deg:
        scratch += [
            pltpu.VMEM((CH,), jnp.float32),
            pltpu.VMEM((RPS,), jnp.float32),
            pltpu.VMEM_SHARED((NPAD,), jnp.float32),
        ]
        out_type.append(jax.ShapeDtypeStruct((NC, NPAD), jnp.float32))

    mesh = plsc.VectorSubcoreMesh(core_axis_name="c", subcore_axis_name="s")

    @functools.partial(pl.kernel, mesh=mesh, out_type=out_type,
                       scratch_types=scratch)
    def k(y_hbm, src_hbm, dst_hbm, agg_out, *rest):
        if compute_deg:
            deg_out, srcidx, dstidx, rows, acc, sem, ones, zrow, dacc = rest
        else:
            srcidx, dstidx, rows, acc, sem = rest
        c = lax.axis_index("c")
        s = lax.axis_index("s")
        wid = s * NC + c
        ebase = wid * EPW

        def zbody(kk, _):
            r = kk // (d // 16)
            i = kk % (d // 16)
            rows[r, pl.ds(i * 16, 16)] = jnp.zeros((16,), jnp.float32)
            return _
        lax.fori_loop(0, CH * (d // 16), zbody, None)
        for t in range(RPS // CH):
            pltpu.sync_copy(rows, acc.at[pl.ds(s * RPS + t * CH, CH)])
        if compute_deg:
            for i in range(CH // 16):
                ones[pl.ds(i * 16, 16)] = jnp.ones((16,), jnp.float32)
            for i in range(RPS // 16):
                zrow[pl.ds(i * 16, 16)] = jnp.zeros((16,), jnp.float32)
            pltpu.sync_copy(zrow, dacc.at[pl.ds(s * RPS, RPS)])

        pltpu.sync_copy(src_hbm.at[pl.ds(ebase, EPW)], srcidx.at[pl.ds(0, EPW)])
        for i in range((SRCPAD - EPW) // 16):
            srcidx[pl.ds(EPW + i * 16, 16)] = jnp.zeros((16,), jnp.int32)
        for i in range(CH // 16):
            dstidx[NFULL, pl.ds(i * 16, 16)] = jnp.full((16,), N, jnp.int32)
        pltpu.sync_copy(dst_hbm.at[pl.ds(ebase + NFULL * CH, TAIL)],
                        dstidx.at[NFULL, pl.ds(0, TAIL)])

        plsc.subcore_barrier()

        def step(j, _):
            @pl.when(j < NFULL)
            def _copy_dst():
                pltpu.sync_copy(dst_hbm.at[pl.ds(ebase + j * CH, CH)],
                                dstidx.at[j])
            pltpu.async_copy(y_hbm.at[srcidx.at[pl.ds(j * CH, CH)]],
                             rows, sem).wait()
            pltpu.sync_copy(rows, acc.at[dstidx.at[j]], add=True)
            if compute_deg:
                pltpu.sync_copy(ones, dacc.at[dstidx.at[j]], add=True)
            return _
        lax.fori_loop(0, NCHUNK, step, None)

        plsc.subcore_barrier()

        pltpu.sync_copy(acc.at[pl.ds(s * RPS, RPS)],
                        agg_out.at[c, pl.ds(s * RPS, RPS)])
        if compute_deg:
            pltpu.sync_copy(dacc.at[pl.ds(s * RPS, RPS)],
                            deg_out.at[c, pl.ds(s * RPS, RPS)])

    return k


_agg_deg128 = _sc_agg(D, True)
_agg128 = _sc_agg(D, False)

_R = 1000


def _row_spec(w):
    return pl.BlockSpec((_R, w), lambda i: (i, 0))


def _const_spec(h, w):
    return pl.BlockSpec((h, w), lambda i: (0, 0))


def _tc_layer_body(relu, aggA, aggB, degA, degB, x, Wl, Wr, bl, br, o_ref):
    deg = jnp.maximum(degA[...] + degB[...], 1.0)
    mean = (aggA[...] + aggB[...]) / deg
    o = (jnp.dot(mean, Wl[...], preferred_element_type=jnp.float32)
         + jnp.dot(x[...], Wr[...], preferred_element_type=jnp.float32)
         + bl[...] + br[...])
    ss = jnp.sum(o * o, axis=-1, keepdims=True)
    o = o * lax.rsqrt(jnp.maximum(ss, 1e-24))
    if relu:
        o = jnp.maximum(o, 0.0)
    o_ref[...] = o


def _tc0(aggA, aggB, degA, degB, x, Wl, Wr, bl, br):
    return pl.pallas_call(
        functools.partial(_tc_layer_body, True),
        grid=(N // _R,),
        in_specs=[_row_spec(D), _row_spec(D), _row_spec(1), _row_spec(1),
                  _row_spec(D), _const_spec(D, D), _const_spec(D, D),
                  _const_spec(1, D), _const_spec(1, D)],
        out_specs=_row_spec(D),
        out_shape=jax.ShapeDtypeStruct((N, D), jnp.float32),
    )(aggA, aggB, degA, degB, x, Wl, Wr, bl, br)


def _tcf_body(aggA, aggB, degA, degB, h, Wlo16, Wro16, blo16, bro16, out_ref):
    deg = jnp.maximum(degA[...] + degB[...], 1.0)
    mean = (aggA[...] + aggB[...]) / deg
    o16 = (jnp.dot(mean, Wlo16[...], preferred_element_type=jnp.float32)
           + jnp.dot(h[...], Wro16[...], preferred_element_type=jnp.float32)
           + blo16[...] + bro16[...])
    o = o16[:, :C]
    ss = jnp.sum(o * o, axis=-1, keepdims=True)
    o = o * lax.rsqrt(jnp.maximum(ss, 1e-24))
    m = jnp.max(o, axis=-1, keepdims=True)
    lse = m + jnp.log(jnp.sum(jnp.exp(o - m), axis=-1, keepdims=True))
    out_ref[...] = o - lse


def _tcf(aggA, aggB, degA, degB, h, Wlo16, Wro16, blo16, bro16):
    return pl.pallas_call(
        _tcf_body,
        grid=(N // _R,),
        in_specs=[_row_spec(D), _row_spec(D), _row_spec(1), _row_spec(1),
                  _row_spec(D), _const_spec(D, 16), _const_spec(D, 16),
                  _const_spec(1, 16), _const_spec(1, 16)],
        out_specs=_row_spec(C),
        out_shape=jax.ShapeDtypeStruct((N, C), jnp.float32),
    )(aggA, aggB, degA, degB, h, Wlo16, Wro16, blo16, bro16)


def kernel(x, edge_index, Wl0, bl0, Wr0, br0, Wl1, bl1, Wr1, br1,
           Wlo, blo, Wro, bro):
    src = edge_index[0]
    dst = edge_index[1]

    agg0, deg = _agg_deg128(x, src, dst)
    degA = deg[0, :N].reshape(N, 1)
    degB = deg[1, :N].reshape(N, 1)

    h1 = _tc0(agg0[0, :N], agg0[1, :N], degA, degB, x,
              Wl0, Wr0, bl0.reshape(1, D), br0.reshape(1, D))

    (agg1,) = _agg128(h1, src, dst)

    h2 = _tc0(agg1[0, :N], agg1[1, :N], degA, degB, h1,
              Wl1, Wr1, bl1.reshape(1, D), br1.reshape(1, D))

    (agg2,) = _agg128(h2, src, dst)

    Wlo16 = jnp.pad(Wlo, ((0, 0), (0, 16 - C)))
    Wro16 = jnp.pad(Wro, ((0, 0), (0, 16 - C)))
    blo16 = jnp.pad(blo, (0, 16 - C)).reshape(1, 16)
    bro16 = jnp.pad(bro, (0, 16 - C)).reshape(1, 16)
    return _tcf(agg2[0, :N], agg2[1, :N], degA, degB, h2,
                Wlo16, Wro16, blo16, bro16)


# speedup vs baseline: 5.7154x; 1.1929x over previous
import functools

import jax
import jax.numpy as jnp
from jax import lax
from jax.experimental import pallas as pl
from jax.experimental.pallas import tpu as pltpu
from jax.experimental.pallas import tpu_sc as plsc

N = 10000
D = 128
E = 320000
C = 2

NC = 2
NS = 16
NW = NC * NS
EPW = E // NW
CH = 128
NFULL = EPW // CH
TAIL = EPW - NFULL * CH
NCHUNK = NFULL + 1
SRCPAD = NCHUNK * CH
NPAD = 10240
RPS = NPAD // NS


def _sc_agg(d, compute_deg):
    scratch = [
        pltpu.VMEM((SRCPAD,), jnp.int32),
        pltpu.VMEM((NCHUNK, CH), jnp.int32),
        pltpu.VMEM((CH, d), jnp.float32),
        pltpu.VMEM_SHARED((NPAD, d), jnp.float32),
        pltpu.SemaphoreType.DMA,
    ]
    out_type = [jax.ShapeDtypeStruct((NC, NPAD, d), jnp.float32)]
    if compute_deg:
        scratch += [
            pltpu.VMEM((CH,), jnp.float32),
            pltpu.VMEM((RPS,), jnp.float32),
            pltpu.VMEM_SHARED((NPAD,), jnp.float32),
        ]
        out_type.append(jax.ShapeDtypeStruct((NC, NPAD), jnp.float32))

    mesh = plsc.VectorSubcoreMesh(core_axis_name="c", subcore_axis_name="s")

    @functools.partial(pl.kernel, mesh=mesh, out_type=out_type,
                       scratch_types=scratch)
    def k(y_hbm, src_hbm, dst_hbm, agg_out, *rest):
        if compute_deg:
            deg_out, srcidx, dstidx, rows, acc, sem, ones, zrow, dacc = rest
        else:
            srcidx, dstidx, rows, acc, sem = rest
        c = lax.axis_index("c")
        s = lax.axis_index("s")
        wid = s * NC + c
        ebase = wid * EPW

        def zbody(kk, _):
            r = kk // (d // 16)
            i = kk % (d // 16)
            rows[r, pl.ds(i * 16, 16)] = jnp.zeros((16,), jnp.float32)
            return _
        lax.fori_loop(0, CH * (d // 16), zbody, None)
        for t in range(RPS // CH):
            pltpu.sync_copy(rows, acc.at[pl.ds(s * RPS + t * CH, CH)])
        if compute_deg:
            for i in range(CH // 16):
                ones[pl.ds(i * 16, 16)] = jnp.ones((16,), jnp.float32)
            for i in range(RPS // 16):
                zrow[pl.ds(i * 16, 16)] = jnp.zeros((16,), jnp.float32)
            pltpu.sync_copy(zrow, dacc.at[pl.ds(s * RPS, RPS)])

        pltpu.sync_copy(src_hbm.at[pl.ds(ebase, EPW)], srcidx.at[pl.ds(0, EPW)])
        for i in range((SRCPAD - EPW) // 16):
            srcidx[pl.ds(EPW + i * 16, 16)] = jnp.zeros((16,), jnp.int32)
        for i in range(CH // 16):
            dstidx[NFULL, pl.ds(i * 16, 16)] = jnp.full((16,), N, jnp.int32)
        pltpu.sync_copy(dst_hbm.at[pl.ds(ebase + NFULL * CH, TAIL)],
                        dstidx.at[NFULL, pl.ds(0, TAIL)])

        plsc.subcore_barrier()

        def step(j, _):
            @pl.when(j < NFULL)
            def _copy_dst():
                pltpu.sync_copy(dst_hbm.at[pl.ds(ebase + j * CH, CH)],
                                dstidx.at[j])
            pltpu.async_copy(y_hbm.at[srcidx.at[pl.ds(j * CH, CH)]],
                             rows, sem).wait()
            pltpu.sync_copy(rows, acc.at[dstidx.at[j]], add=True)
            if compute_deg:
                pltpu.sync_copy(ones, dacc.at[dstidx.at[j]], add=True)
            return _
        lax.fori_loop(0, NCHUNK, step, None)

        plsc.subcore_barrier()

        pltpu.sync_copy(acc.at[pl.ds(s * RPS, RPS)],
                        agg_out.at[c, pl.ds(s * RPS, RPS)])
        if compute_deg:
            pltpu.sync_copy(dacc.at[pl.ds(s * RPS, RPS)],
                            deg_out.at[c, pl.ds(s * RPS, RPS)])

    return k


def _sc_agg2():
    scratch = [
        pltpu.VMEM((SRCPAD,), jnp.int32),
        pltpu.VMEM((NCHUNK, CH), jnp.int32),
        pltpu.VMEM((CH,), jnp.float32),
        pltpu.VMEM((CH,), jnp.float32),
        pltpu.VMEM((RPS,), jnp.float32),
        pltpu.VMEM_SHARED((NPAD,), jnp.float32),
        pltpu.VMEM_SHARED((NPAD,), jnp.float32),
        pltpu.SemaphoreType.DMA,
        pltpu.SemaphoreType.DMA,
    ]
    out_type = [jax.ShapeDtypeStruct((NC, NPAD), jnp.float32),
                jax.ShapeDtypeStruct((NC, NPAD), jnp.float32)]
    mesh = plsc.VectorSubcoreMesh(core_axis_name="c", subcore_axis_name="s")

    @functools.partial(pl.kernel, mesh=mesh, out_type=out_type,
                       scratch_types=scratch)
    def k(p0_hbm, p1_hbm, src_hbm, dst_hbm, out0, out1,
          srcidx, dstidx, vals0, vals1, zrow, acc0, acc1, sem0, sem1):
        c = lax.axis_index("c")
        s = lax.axis_index("s")
        wid = s * NC + c
        ebase = wid * EPW

        for i in range(RPS // 16):
            zrow[pl.ds(i * 16, 16)] = jnp.zeros((16,), jnp.float32)
        pltpu.sync_copy(zrow, acc0.at[pl.ds(s * RPS, RPS)])
        pltpu.sync_copy(zrow, acc1.at[pl.ds(s * RPS, RPS)])

        pltpu.sync_copy(src_hbm.at[pl.ds(ebase, EPW)], srcidx.at[pl.ds(0, EPW)])
        for i in range((SRCPAD - EPW) // 16):
            srcidx[pl.ds(EPW + i * 16, 16)] = jnp.zeros((16,), jnp.int32)
        for i in range(CH // 16):
            dstidx[NFULL, pl.ds(i * 16, 16)] = jnp.full((16,), N, jnp.int32)
        pltpu.sync_copy(dst_hbm.at[pl.ds(ebase + NFULL * CH, TAIL)],
                        dstidx.at[NFULL, pl.ds(0, TAIL)])

        plsc.subcore_barrier()

        def step(j, _):
            @pl.when(j < NFULL)
            def _copy_dst():
                pltpu.sync_copy(dst_hbm.at[pl.ds(ebase + j * CH, CH)],
                                dstidx.at[j])
            cp0 = pltpu.async_copy(p0_hbm.at[srcidx.at[pl.ds(j * CH, CH)]],
                                   vals0, sem0)
            cp1 = pltpu.async_copy(p1_hbm.at[srcidx.at[pl.ds(j * CH, CH)]],
                                   vals1, sem1)
            cp0.wait()
            cp1.wait()
            pltpu.sync_copy(vals0, acc0.at[dstidx.at[j]], add=True)
            pltpu.sync_copy(vals1, acc1.at[dstidx.at[j]], add=True)
            return _
        lax.fori_loop(0, NCHUNK, step, None)

        plsc.subcore_barrier()

        pltpu.sync_copy(acc0.at[pl.ds(s * RPS, RPS)],
                        out0.at[c, pl.ds(s * RPS, RPS)])
        pltpu.sync_copy(acc1.at[pl.ds(s * RPS, RPS)],
                        out1.at[c, pl.ds(s * RPS, RPS)])

    return k


_agg_deg128 = _sc_agg(D, True)
_agg128 = _sc_agg(D, False)
_agg2 = _sc_agg2()

_R = 1000


def _row_spec(w):
    return pl.BlockSpec((_R, w), lambda i: (i, 0))


def _const_spec(h, w):
    return pl.BlockSpec((h, w), lambda i: (0, 0))


def _tc_layer_body(relu, aggA, aggB, degA, degB, x, Wl, Wr, bl, br, o_ref):
    deg = jnp.maximum(degA[...] + degB[...], 1.0)
    mean = (aggA[...] + aggB[...]) / deg
    o = (jnp.dot(mean, Wl[...], preferred_element_type=jnp.float32)
         + jnp.dot(x[...], Wr[...], preferred_element_type=jnp.float32)
         + bl[...] + br[...])
    ss = jnp.sum(o * o, axis=-1, keepdims=True)
    o = o * lax.rsqrt(jnp.maximum(ss, 1e-24))
    if relu:
        o = jnp.maximum(o, 0.0)
    o_ref[...] = o


def _tc0(aggA, aggB, degA, degB, x, Wl, Wr, bl, br):
    return pl.pallas_call(
        functools.partial(_tc_layer_body, True),
        grid=(N // _R,),
        in_specs=[_row_spec(D), _row_spec(D), _row_spec(1), _row_spec(1),
                  _row_spec(D), _const_spec(D, D), _const_spec(D, D),
                  _const_spec(1, D), _const_spec(1, D)],
        out_specs=_row_spec(D),
        out_shape=jax.ShapeDtypeStruct((N, D), jnp.float32),
    )(aggA, aggB, degA, degB, x, Wl, Wr, bl, br)


def _tc1_body(aggA, aggB, degA, degB, x, Wl, Wr, bl, br, Wlo16,
              h_ref, p_ref):
    deg = jnp.maximum(degA[...] + degB[...], 1.0)
    mean = (aggA[...] + aggB[...]) / deg
    o = (jnp.dot(mean, Wl[...], preferred_element_type=jnp.float32)
         + jnp.dot(x[...], Wr[...], preferred_element_type=jnp.float32)
         + bl[...] + br[...])
    ss = jnp.sum(o * o, axis=-1, keepdims=True)
    o = o * lax.rsqrt(jnp.maximum(ss, 1e-24))
    o = jnp.maximum(o, 0.0)
    h_ref[...] = o
    p_ref[...] = jnp.dot(o, Wlo16[...], preferred_element_type=jnp.float32)


def _tc1(aggA, aggB, degA, degB, x, Wl, Wr, bl, br, Wlo16):
    return pl.pallas_call(
        _tc1_body,
        grid=(N // _R,),
        in_specs=[_row_spec(D), _row_spec(D), _row_spec(1), _row_spec(1),
                  _row_spec(D), _const_spec(D, D), _const_spec(D, D),
                  _const_spec(1, D), _const_spec(1, D), _const_spec(D, 16)],
        out_specs=[_row_spec(D), _row_spec(16)],
        out_shape=[jax.ShapeDtypeStruct((N, D), jnp.float32),
                   jax.ShapeDtypeStruct((N, 16), jnp.float32)],
    )(aggA, aggB, degA, degB, x, Wl, Wr, bl, br, Wlo16)


def _tcf_body(a0A, a0B, a1A, a1B, degA, degB, h, Wro16, blo16, bro16,
              out_ref):
    deg = jnp.maximum(degA[...] + degB[...], 1.0)
    mean = jnp.concatenate(
        [(a0A[...] + a0B[...]) / deg, (a1A[...] + a1B[...]) / deg], axis=1)
    o16 = (jnp.dot(h[...], Wro16[...], preferred_element_type=jnp.float32)
           + blo16[...] + bro16[...])
    o = o16[:, :C] + mean
    ss = jnp.sum(o * o, axis=-1, keepdims=True)
    o = o * lax.rsqrt(jnp.maximum(ss, 1e-24))
    m = jnp.max(o, axis=-1, keepdims=True)
    lse = m + jnp.log(jnp.sum(jnp.exp(o - m), axis=-1, keepdims=True))
    out_ref[...] = o - lse


def _tcf(a0A, a0B, a1A, a1B, degA, degB, h, Wro16, blo16, bro16):
    return pl.pallas_call(
        _tcf_body,
        grid=(N // _R,),
        in_specs=[_row_spec(1), _row_spec(1), _row_spec(1), _row_spec(1),
                  _row_spec(1), _row_spec(1),
                  _row_spec(D), _const_spec(D, 16),
                  _const_spec(1, 16), _const_spec(1, 16)],
        out_specs=_row_spec(C),
        out_shape=jax.ShapeDtypeStruct((N, C), jnp.float32),
    )(a0A, a0B, a1A, a1B, degA, degB, h, Wro16, blo16, bro16)


def kernel(x, edge_index, Wl0, bl0, Wr0, br0, Wl1, bl1, Wr1, br1,
           Wlo, blo, Wro, bro):
    src = edge_index[0]
    dst = edge_index[1]

    agg0, deg = _agg_deg128(x, src, dst)
    degA = deg[0, :N].reshape(N, 1)
    degB = deg[1, :N].reshape(N, 1)

    h1 = _tc0(agg0[0, :N], agg0[1, :N], degA, degB, x,
              Wl0, Wr0, bl0.reshape(1, D), br0.reshape(1, D))

    (agg1,) = _agg128(h1, src, dst)

    Wlo16 = jnp.pad(Wlo, ((0, 0), (0, 16 - C)))
    h2, p16 = _tc1(agg1[0, :N], agg1[1, :N], degA, degB, h1,
                   Wl1, Wr1, bl1.reshape(1, D), br1.reshape(1, D), Wlo16)

    agg2c0, agg2c1 = _agg2(p16[:, 0], p16[:, 1], src, dst)

    Wro16 = jnp.pad(Wro, ((0, 0), (0, 16 - C)))
    blo16 = jnp.pad(blo, (0, 16 - C)).reshape(1, 16)
    bro16 = jnp.pad(bro, (0, 16 - C)).reshape(1, 16)
    return _tcf(agg2c0[0, :N].reshape(N, 1), agg2c0[1, :N].reshape(N, 1),
                agg2c1[0, :N].reshape(N, 1), agg2c1[1, :N].reshape(N, 1),
                degA, degB, h2, Wro16, blo16, bro16)
